# Initial kernel scaffold; baseline (speedup 1.0000x reference)
import jax
import jax.numpy as jnp
from jax.experimental import pallas as pl


def kernel(x, edge_index, edge_attr, W1, b1, W2, b2, Wd1, bd1, Wd2, bd2):
    raise NotImplementedError("write your pallas kernel here")


# speedup vs baseline: 10.7762x; 10.7762x over previous
import functools

import jax
import jax.numpy as jnp
from jax import lax
from jax.experimental import pallas as pl
from jax.experimental.pallas import tpu as pltpu
from jax.experimental.pallas import tpu_sc as plsc

NC = 2
NS = 16
NW = NC * NS
CHUNK = 128

f32 = jnp.float32


def _sc_mesh():
    return plsc.VectorSubcoreMesh(core_axis_name="c", subcore_axis_name="s")


_SC_PARAMS = pltpu.CompilerParams(use_tc_tiling_on_sc=False)


@functools.lru_cache(maxsize=None)
def _make_sc_agg(N, E, F):
    assert E % CHUNK == 0 and N % NS == 0
    n_chunks = E // CHUNK
    iters = (n_chunks + NW - 1) // NW
    rpt = N // NS

    @functools.partial(
        pl.kernel,
        mesh=_sc_mesh(),
        out_type=jax.ShapeDtypeStruct((NC, NS, rpt, F), f32),
        scratch_types=[
            pltpu.VMEM((CHUNK,), jnp.int32),
            pltpu.VMEM((CHUNK,), jnp.int32),
            pltpu.VMEM((CHUNK, F), f32),
            pltpu.VMEM_SHARED((N, F), f32),
            pltpu.SemaphoreType.DMA,
        ],
        compiler_params=_SC_PARAMS,
    )
    def k(g_hbm, src_hbm, dst_hbm, z_hbm, out_hbm, idx_s, idx_d, rows, acc, sem):
        cid = lax.axis_index("c")
        sid = lax.axis_index("s")
        wid = sid * NC + cid
        r0 = sid * rpt
        pltpu.sync_copy(z_hbm.at[sid], acc.at[pl.ds(r0, rpt)])
        plsc.subcore_barrier()

        def body(i, carry):
            c = wid + i * NW

            @pl.when(c < n_chunks)
            def _():
                base = c * CHUNK
                pltpu.sync_copy(src_hbm.at[pl.ds(base, CHUNK)], idx_s)
                pltpu.sync_copy(dst_hbm.at[pl.ds(base, CHUNK)], idx_d)
                pltpu.async_copy(g_hbm.at[idx_s], rows, sem).wait()
                pltpu.sync_copy(rows, acc.at[idx_d], add=True)

            return carry

        lax.fori_loop(0, iters, body, 0)
        plsc.subcore_barrier()
        pltpu.sync_copy(acc.at[pl.ds(r0, rpt)], out_hbm.at[cid, sid])

    return k


@functools.lru_cache(maxsize=None)
def _make_sc_na(N, E):
    F = 32
    assert E % CHUNK == 0 and N % NS == 0
    n_chunks = E // CHUNK
    iters = (n_chunks + NW - 1) // NW
    rpt = N // NS

    @functools.partial(
        pl.kernel,
        mesh=_sc_mesh(),
        out_type=jax.ShapeDtypeStruct((NC, 2, NS, rpt, F), f32),
        scratch_types=[
            pltpu.VMEM((CHUNK,), jnp.int32),
            pltpu.VMEM((CHUNK,), jnp.int32),
            pltpu.VMEM((CHUNK, F), f32),
            pltpu.VMEM_SHARED((N, F), f32),
            pltpu.VMEM_SHARED((N, F), f32),
        ],
        compiler_params=_SC_PARAMS,
    )
    def k(ea_hbm, src_hbm, dst_hbm, z_hbm, out_hbm, idx_s, idx_d, rows,
          acc_s, acc_d):
        cid = lax.axis_index("c")
        sid = lax.axis_index("s")
        wid = sid * NC + cid
        r0 = sid * rpt
        pltpu.sync_copy(z_hbm.at[sid], acc_s.at[pl.ds(r0, rpt)])
        pltpu.sync_copy(z_hbm.at[sid], acc_d.at[pl.ds(r0, rpt)])
        plsc.subcore_barrier()

        def body(i, carry):
            c = wid + i * NW

            @pl.when(c < n_chunks)
            def _():
                base = c * CHUNK
                pltpu.sync_copy(src_hbm.at[pl.ds(base, CHUNK)], idx_s)
                pltpu.sync_copy(dst_hbm.at[pl.ds(base, CHUNK)], idx_d)
                pltpu.sync_copy(ea_hbm.at[pl.ds(base, CHUNK)], rows)
                pltpu.sync_copy(rows, acc_s.at[idx_s], add=True)
                pltpu.sync_copy(rows, acc_d.at[idx_d], add=True)

            return carry

        lax.fori_loop(0, iters, body, 0)
        plsc.subcore_barrier()
        pltpu.sync_copy(acc_s.at[pl.ds(r0, rpt)], out_hbm.at[cid, 0, sid])
        pltpu.sync_copy(acc_d.at[pl.ds(r0, rpt)], out_hbm.at[cid, 1, sid])

    return k


BN = 2000


def _tc1(accs, x, w1a, w1b):
    N, DF = x.shape
    H = w1a.shape[1]
    grid = (N // BN,)

    def body(a_ref, x_ref, wa_ref, wb_ref, g_ref, dis_ref):
        a = a_ref[...]
        attr = a[0, 0, :, 0:16] + a[0, 1, :, 0:16] + a[1, 0, :, 0:16] + a[1, 1, :, 0:16]
        cnt = a[0, 0, :, 16:17] + a[0, 1, :, 16:17] + a[1, 0, :, 16:17] + a[1, 1, :, 16:17]
        degd = a[0, 1, :, 16:17] + a[1, 1, :, 16:17]
        na = attr / (cnt + 1e-8)
        dis = lax.rsqrt(degd + 1.0)
        h = (jnp.dot(x_ref[...], wa_ref[...], preferred_element_type=f32)
             + jnp.dot(na, wb_ref[...], preferred_element_type=f32))
        g_ref[...] = h * dis
        dis_ref[...] = dis

    return pl.pallas_call(
        body,
        grid=grid,
        in_specs=[
            pl.BlockSpec((NC, 2, BN, 32), lambda i: (0, 0, i, 0)),
            pl.BlockSpec((BN, DF), lambda i: (i, 0)),
            pl.BlockSpec((DF, H), lambda i: (0, 0)),
            pl.BlockSpec((16, H), lambda i: (0, 0)),
        ],
        out_specs=[
            pl.BlockSpec((BN, H), lambda i: (i, 0)),
            pl.BlockSpec((BN, 1), lambda i: (i, 0)),
        ],
        out_shape=[
            jax.ShapeDtypeStruct((N, H), f32),
            jax.ShapeDtypeStruct((N, 1), f32),
        ],
    )(accs, x, w1a, w1b)


def _tc_mid(aggs, g, dis, b, w):
    N, F = g.shape
    Fo = w.shape[1]
    grid = (N // BN,)

    def body(a_ref, g_ref, dis_ref, b_ref, w_ref, out_ref):
        a = a_ref[...]
        dis = dis_ref[...]
        s = dis * (a[0] + a[1] + g_ref[...]) + b_ref[...]
        h = jnp.maximum(s, 0.0)
        out_ref[...] = jnp.dot(h, w_ref[...], preferred_element_type=f32) * dis

    return pl.pallas_call(
        body,
        grid=grid,
        in_specs=[
            pl.BlockSpec((NC, BN, F), lambda i: (0, i, 0)),
            pl.BlockSpec((BN, F), lambda i: (i, 0)),
            pl.BlockSpec((BN, 1), lambda i: (i, 0)),
            pl.BlockSpec((1, F), lambda i: (0, 0)),
            pl.BlockSpec((F, Fo), lambda i: (0, 0)),
        ],
        out_specs=pl.BlockSpec((BN, Fo), lambda i: (i, 0)),
        out_shape=jax.ShapeDtypeStruct((N, Fo), f32),
    )(aggs, g, dis, b, w)


def _tc_fin(aggs, g, dis, b):
    N, F = g.shape
    grid = (N // BN,)

    def body(a_ref, g_ref, dis_ref, b_ref, out_ref):
        a = a_ref[...]
        out_ref[...] = dis_ref[...] * (a[0] + a[1] + g_ref[...]) + b_ref[...]

    return pl.pallas_call(
        body,
        grid=grid,
        in_specs=[
            pl.BlockSpec((NC, BN, F), lambda i: (0, i, 0)),
            pl.BlockSpec((BN, F), lambda i: (i, 0)),
            pl.BlockSpec((BN, 1), lambda i: (i, 0)),
            pl.BlockSpec((1, F), lambda i: (0, 0)),
        ],
        out_specs=pl.BlockSpec((BN, F), lambda i: (i, 0)),
        out_shape=jax.ShapeDtypeStruct((N, F), f32),
    )(aggs, g, dis, b)


def kernel(x, edge_index, edge_attr, W1, b1, W2, b2, Wd1, bd1, Wd2, bd2):
    N, DF = x.shape
    E = edge_index.shape[1]
    DE = edge_attr.shape[1]
    src = edge_index[0]
    dst = edge_index[1]

    ea_plus = jnp.concatenate(
        [edge_attr, jnp.ones((E, 1), f32), jnp.zeros((E, 32 - DE - 1), f32)],
        axis=1)

    rpt = N // NS
    na_acc = _make_sc_na(N, E)(
        ea_plus, src, dst, jnp.zeros((NS, rpt, 32), f32)
    ).reshape(NC, 2, N, 32)
    g1, dis = _tc1(na_acc, x, W1[:DF], W1[DF:])

    def conv_agg(g):
        F = g.shape[1]
        return _make_sc_agg(N, E, F)(
            g, src, dst, jnp.zeros((NS, rpt, F), f32)
        ).reshape(NC, N, F)

    g2 = _tc_mid(conv_agg(g1), g1, dis, b1.reshape(1, -1), W2)
    g3 = _tc_mid(conv_agg(g2), g2, dis, b2.reshape(1, -1), Wd1)
    g4 = _tc_mid(conv_agg(g3), g3, dis, bd1.reshape(1, -1), Wd2)
    return _tc_fin(conv_agg(g4), g4, dis, bd2.reshape(1, -1))


# speedup vs baseline: 14.2959x; 1.3266x over previous
import functools

import jax
import jax.numpy as jnp
from jax import lax
from jax.experimental import pallas as pl
from jax.experimental.pallas import tpu as pltpu
from jax.experimental.pallas import tpu_sc as plsc

NC = 2
NS = 16
NW = NC * NS
CHUNK = 80

f32 = jnp.float32


def _sc_mesh():
    return plsc.VectorSubcoreMesh(core_axis_name="c", subcore_axis_name="s")


_SC_PARAMS = pltpu.CompilerParams(use_tc_tiling_on_sc=False)


@functools.lru_cache(maxsize=None)
def _make_sc_agg(N, E, F, CHUNK=CHUNK):
    assert N % NS == 0
    assert E % (NW * CHUNK) == 0
    nch = E // (NW * CHUNK)
    rpt = N // NS

    @functools.partial(
        pl.kernel,
        mesh=_sc_mesh(),
        out_type=jax.ShapeDtypeStruct((NC, NS, rpt, F), f32),
        scratch_types=[
            pltpu.VMEM((nch, CHUNK), jnp.int32),
            pltpu.VMEM((nch, CHUNK), jnp.int32),
            pltpu.VMEM((CHUNK, F), f32),
            pltpu.VMEM((CHUNK, F), f32),
            pltpu.SemaphoreType.DMA,
            pltpu.SemaphoreType.DMA,
            pltpu.VMEM_SHARED((N, F), f32),
        ],
        compiler_params=_SC_PARAMS,
    )
    def k(g_hbm, src_hbm, dst_hbm, z_hbm, out_hbm,
          idxs, idxd, rows0, rows1, sem0, sem1, acc):
        cid = lax.axis_index("c")
        sid = lax.axis_index("s")
        wid = sid * NC + cid
        r0 = sid * rpt
        c0 = wid * nch
        pltpu.sync_copy(z_hbm.at[sid], acc.at[pl.ds(r0, rpt)])
        pltpu.sync_copy(src_hbm.at[pl.ds(c0, nch)], idxs)
        pltpu.sync_copy(dst_hbm.at[pl.ds(c0, nch)], idxd)
        plsc.subcore_barrier()

        pltpu.async_copy(g_hbm.at[idxs.at[0]], rows0, sem0)

        def body(k2, carry):
            i0 = 2 * k2
            i1 = i0 + 1
            pltpu.make_async_copy(g_hbm.at[idxs.at[i0]], rows0, sem0).wait()
            pltpu.async_copy(g_hbm.at[idxs.at[i1]], rows1, sem1)
            pltpu.sync_copy(rows0, acc.at[idxd.at[i0]], add=True)
            pltpu.make_async_copy(g_hbm.at[idxs.at[i1]], rows1, sem1).wait()

            @pl.when(i0 + 2 < nch)
            def _():
                pltpu.async_copy(g_hbm.at[idxs.at[i0 + 2]], rows0, sem0)

            pltpu.sync_copy(rows1, acc.at[idxd.at[i1]], add=True)
            return carry

        lax.fori_loop(0, nch // 2, body, 0)
        if nch % 2:
            i = nch - 1
            pltpu.make_async_copy(g_hbm.at[idxs.at[i]], rows0, sem0).wait()
            pltpu.sync_copy(rows0, acc.at[idxd.at[i]], add=True)
        plsc.subcore_barrier()
        pltpu.sync_copy(acc.at[pl.ds(r0, rpt)], out_hbm.at[cid, sid])

    return k


@functools.lru_cache(maxsize=None)
def _make_sc_na(N, E):
    F = 32
    assert N % NS == 0
    assert E % (NW * CHUNK) == 0
    nch = E // (NW * CHUNK)
    rpt = N // NS

    @functools.partial(
        pl.kernel,
        mesh=_sc_mesh(),
        out_type=jax.ShapeDtypeStruct((NC, 2, NS, rpt, F), f32),
        scratch_types=[
            pltpu.VMEM((nch, CHUNK), jnp.int32),
            pltpu.VMEM((nch, CHUNK), jnp.int32),
            pltpu.VMEM((CHUNK, F), f32),
            pltpu.VMEM((CHUNK, F), f32),
            pltpu.SemaphoreType.DMA,
            pltpu.SemaphoreType.DMA,
            pltpu.VMEM_SHARED((N, F), f32),
            pltpu.VMEM_SHARED((N, F), f32),
        ],
        compiler_params=_SC_PARAMS,
    )
    def k(ea_hbm, src_hbm, dst_hbm, z_hbm, out_hbm,
          idxs, idxd, rows0, rows1, sem0, sem1, acc_s, acc_d):
        cid = lax.axis_index("c")
        sid = lax.axis_index("s")
        wid = sid * NC + cid
        r0 = sid * rpt
        c0 = wid * nch
        e0 = c0 * CHUNK
        pltpu.sync_copy(z_hbm.at[sid], acc_s.at[pl.ds(r0, rpt)])
        pltpu.sync_copy(z_hbm.at[sid], acc_d.at[pl.ds(r0, rpt)])
        pltpu.sync_copy(src_hbm.at[pl.ds(c0, nch)], idxs)
        pltpu.sync_copy(dst_hbm.at[pl.ds(c0, nch)], idxd)
        plsc.subcore_barrier()

        pltpu.async_copy(ea_hbm.at[pl.ds(e0, CHUNK)], rows0, sem0)

        def body(k2, carry):
            i0 = 2 * k2
            i1 = i0 + 1
            pltpu.make_async_copy(
                ea_hbm.at[pl.ds(e0 + i0 * CHUNK, CHUNK)], rows0, sem0).wait()
            pltpu.async_copy(
                ea_hbm.at[pl.ds(e0 + i1 * CHUNK, CHUNK)], rows1, sem1)
            pltpu.sync_copy(rows0, acc_s.at[idxs.at[i0]], add=True)
            pltpu.sync_copy(rows0, acc_d.at[idxd.at[i0]], add=True)
            pltpu.make_async_copy(
                ea_hbm.at[pl.ds(e0 + i1 * CHUNK, CHUNK)], rows1, sem1).wait()

            @pl.when(i0 + 2 < nch)
            def _():
                pltpu.async_copy(
                    ea_hbm.at[pl.ds(e0 + (i0 + 2) * CHUNK, CHUNK)], rows0, sem0)

            pltpu.sync_copy(rows1, acc_s.at[idxs.at[i1]], add=True)
            pltpu.sync_copy(rows1, acc_d.at[idxd.at[i1]], add=True)
            return carry

        lax.fori_loop(0, nch // 2, body, 0)
        if nch % 2:
            i = nch - 1
            pltpu.make_async_copy(
                ea_hbm.at[pl.ds(e0 + i * CHUNK, CHUNK)], rows0, sem0).wait()
            pltpu.sync_copy(rows0, acc_s.at[idxs.at[i]], add=True)
            pltpu.sync_copy(rows0, acc_d.at[idxd.at[i]], add=True)
        plsc.subcore_barrier()
        pltpu.sync_copy(acc_s.at[pl.ds(r0, rpt)], out_hbm.at[cid, 0, sid])
        pltpu.sync_copy(acc_d.at[pl.ds(r0, rpt)], out_hbm.at[cid, 1, sid])

    return k


BN = 2000


def _tc1(accs, x, w1a, w1b):
    N, DF = x.shape
    H = w1a.shape[1]
    grid = (N // BN,)

    def body(a_ref, x_ref, wa_ref, wb_ref, g_ref, dis_ref):
        a = a_ref[...]
        attr = a[0, 0, :, 0:16] + a[0, 1, :, 0:16] + a[1, 0, :, 0:16] + a[1, 1, :, 0:16]
        cnt = a[0, 0, :, 16:17] + a[0, 1, :, 16:17] + a[1, 0, :, 16:17] + a[1, 1, :, 16:17]
        degd = a[0, 1, :, 16:17] + a[1, 1, :, 16:17]
        na = attr / (cnt + 1e-8)
        dis = lax.rsqrt(degd + 1.0)
        h = (jnp.dot(x_ref[...], wa_ref[...], preferred_element_type=f32)
             + jnp.dot(na, wb_ref[...], preferred_element_type=f32))
        g_ref[...] = h * dis
        dis_ref[...] = dis

    return pl.pallas_call(
        body,
        grid=grid,
        in_specs=[
            pl.BlockSpec((NC, 2, BN, 32), lambda i: (0, 0, i, 0)),
            pl.BlockSpec((BN, DF), lambda i: (i, 0)),
            pl.BlockSpec((DF, H), lambda i: (0, 0)),
            pl.BlockSpec((16, H), lambda i: (0, 0)),
        ],
        out_specs=[
            pl.BlockSpec((BN, H), lambda i: (i, 0)),
            pl.BlockSpec((BN, 1), lambda i: (i, 0)),
        ],
        out_shape=[
            jax.ShapeDtypeStruct((N, H), f32),
            jax.ShapeDtypeStruct((N, 1), f32),
        ],
    )(accs, x, w1a, w1b)


def _tc_mid(aggs, g, dis, b, w):
    N, F = g.shape
    Fo = w.shape[1]
    grid = (N // BN,)

    def body(a_ref, g_ref, dis_ref, b_ref, w_ref, out_ref):
        a = a_ref[...]
        dis = dis_ref[...]
        s = dis * (a[0] + a[1] + g_ref[...]) + b_ref[...]
        h = jnp.maximum(s, 0.0)
        out_ref[...] = jnp.dot(h, w_ref[...], preferred_element_type=f32) * dis

    return pl.pallas_call(
        body,
        grid=grid,
        in_specs=[
            pl.BlockSpec((NC, BN, F), lambda i: (0, i, 0)),
            pl.BlockSpec((BN, F), lambda i: (i, 0)),
            pl.BlockSpec((BN, 1), lambda i: (i, 0)),
            pl.BlockSpec((1, F), lambda i: (0, 0)),
            pl.BlockSpec((F, Fo), lambda i: (0, 0)),
        ],
        out_specs=pl.BlockSpec((BN, Fo), lambda i: (i, 0)),
        out_shape=jax.ShapeDtypeStruct((N, Fo), f32),
    )(aggs, g, dis, b, w)


def _tc_fin(aggs, g, dis, b):
    N, F = g.shape
    grid = (N // BN,)

    def body(a_ref, g_ref, dis_ref, b_ref, out_ref):
        a = a_ref[...]
        out_ref[...] = dis_ref[...] * (a[0] + a[1] + g_ref[...]) + b_ref[...]

    return pl.pallas_call(
        body,
        grid=grid,
        in_specs=[
            pl.BlockSpec((NC, BN, F), lambda i: (0, i, 0)),
            pl.BlockSpec((BN, F), lambda i: (i, 0)),
            pl.BlockSpec((BN, 1), lambda i: (i, 0)),
            pl.BlockSpec((1, F), lambda i: (0, 0)),
        ],
        out_specs=pl.BlockSpec((BN, F), lambda i: (i, 0)),
        out_shape=jax.ShapeDtypeStruct((N, F), f32),
    )(aggs, g, dis, b)


def kernel(x, edge_index, edge_attr, W1, b1, W2, b2, Wd1, bd1, Wd2, bd2):
    N, DF = x.shape
    E = edge_index.shape[1]
    DE = edge_attr.shape[1]
    src = edge_index[0]
    dst = edge_index[1]

    ea_plus = jnp.concatenate(
        [edge_attr, jnp.ones((E, 1), f32), jnp.zeros((E, 32 - DE - 1), f32)],
        axis=1)

    rpt = N // NS
    src2d = src.reshape(E // CHUNK, CHUNK)
    dst2d = dst.reshape(E // CHUNK, CHUNK)
    src2dh = src.reshape(E // (CHUNK // 2), CHUNK // 2)
    dst2dh = dst.reshape(E // (CHUNK // 2), CHUNK // 2)
    na_acc = _make_sc_na(N, E)(
        ea_plus, src2d, dst2d, jnp.zeros((NS, rpt, 32), f32)
    ).reshape(NC, 2, N, 32)
    g1, dis = _tc1(na_acc, x, W1[:DF], W1[DF:])

    def conv_agg(g):
        F = g.shape[1]
        ch = CHUNK if F <= 128 else CHUNK // 2
        s2, d2 = (src2d, dst2d) if ch == CHUNK else (src2dh, dst2dh)
        return _make_sc_agg(N, E, F, ch)(
            g, s2, d2, jnp.zeros((NS, rpt, F), f32)
        ).reshape(NC, N, F)

    g2 = _tc_mid(conv_agg(g1), g1, dis, b1.reshape(1, -1), W2)
    g3 = _tc_mid(conv_agg(g2), g2, dis, b2.reshape(1, -1), Wd1)
    g4 = _tc_mid(conv_agg(g3), g3, dis, bd1.reshape(1, -1), Wd2)
    return _tc_fin(conv_agg(g4), g4, dis, bd2.reshape(1, -1))


# speedup vs baseline: 16.6732x; 1.1663x over previous
import functools

import jax
import jax.numpy as jnp
from jax import lax
from jax.experimental import pallas as pl
from jax.experimental.pallas import tpu as pltpu
from jax.experimental.pallas import tpu_sc as plsc

NC = 2
NS = 16
NW = NC * NS
CHUNK = 80

f32 = jnp.float32


def _sc_mesh():
    return plsc.VectorSubcoreMesh(core_axis_name="c", subcore_axis_name="s")


_SC_PARAMS = pltpu.CompilerParams(use_tc_tiling_on_sc=False)


@functools.lru_cache(maxsize=None)
def _make_sc_agg(N, E, F, CHUNK=CHUNK):
    assert N % NS == 0
    assert E % (NW * CHUNK) == 0
    nch = E // (NW * CHUNK)
    rpt = N // NS

    @functools.partial(
        pl.kernel,
        mesh=_sc_mesh(),
        out_type=jax.ShapeDtypeStruct((NC, NS, rpt, F), f32),
        scratch_types=[
            pltpu.VMEM((nch, CHUNK), jnp.int32),
            pltpu.VMEM((nch, CHUNK), jnp.int32),
            pltpu.VMEM((CHUNK, F), f32),
            pltpu.VMEM((CHUNK, F), f32),
            pltpu.SemaphoreType.DMA,
            pltpu.SemaphoreType.DMA,
            pltpu.VMEM_SHARED((N, F), f32),
        ],
        compiler_params=_SC_PARAMS,
    )
    def k(g_hbm, src_hbm, dst_hbm, z_hbm, out_hbm,
          idxs, idxd, rows0, rows1, sem0, sem1, acc):
        cid = lax.axis_index("c")
        sid = lax.axis_index("s")
        wid = sid * NC + cid
        r0 = sid * rpt
        c0 = wid * nch
        pltpu.sync_copy(z_hbm.at[sid], acc.at[pl.ds(r0, rpt)])
        pltpu.sync_copy(src_hbm.at[pl.ds(c0, nch)], idxs)
        pltpu.sync_copy(dst_hbm.at[pl.ds(c0, nch)], idxd)
        plsc.subcore_barrier()

        pltpu.async_copy(g_hbm.at[idxs.at[0]], rows0, sem0)

        def body(k2, carry):
            i0 = 2 * k2
            i1 = i0 + 1
            pltpu.make_async_copy(g_hbm.at[idxs.at[i0]], rows0, sem0).wait()
            pltpu.async_copy(g_hbm.at[idxs.at[i1]], rows1, sem1)
            pltpu.sync_copy(rows0, acc.at[idxd.at[i0]], add=True)
            pltpu.make_async_copy(g_hbm.at[idxs.at[i1]], rows1, sem1).wait()

            @pl.when(i0 + 2 < nch)
            def _():
                pltpu.async_copy(g_hbm.at[idxs.at[i0 + 2]], rows0, sem0)

            pltpu.sync_copy(rows1, acc.at[idxd.at[i1]], add=True)
            return carry

        lax.fori_loop(0, nch // 2, body, 0)
        if nch % 2:
            i = nch - 1
            pltpu.make_async_copy(g_hbm.at[idxs.at[i]], rows0, sem0).wait()
            pltpu.sync_copy(rows0, acc.at[idxd.at[i]], add=True)
        plsc.subcore_barrier()
        pltpu.sync_copy(acc.at[pl.ds(r0, rpt)], out_hbm.at[cid, sid])

    return k


@functools.lru_cache(maxsize=None)
def _make_sc_na(N, E):
    F = 32
    assert N % NS == 0
    assert E % (NW * CHUNK) == 0
    nch = E // (NW * CHUNK)
    rpt = N // NS

    @functools.partial(
        pl.kernel,
        mesh=_sc_mesh(),
        out_type=jax.ShapeDtypeStruct((NC, 2, NS, rpt, F), f32),
        scratch_types=[
            pltpu.VMEM((nch, CHUNK), jnp.int32),
            pltpu.VMEM((nch, CHUNK), jnp.int32),
            pltpu.VMEM((CHUNK, F), f32),
            pltpu.VMEM((CHUNK, F), f32),
            pltpu.SemaphoreType.DMA,
            pltpu.SemaphoreType.DMA,
            pltpu.VMEM_SHARED((N, F), f32),
            pltpu.VMEM_SHARED((N, F), f32),
        ],
        compiler_params=_SC_PARAMS,
    )
    def k(ea_hbm, src_hbm, dst_hbm, z_hbm, out_hbm,
          idxs, idxd, rows0, rows1, sem0, sem1, acc_s, acc_d):
        cid = lax.axis_index("c")
        sid = lax.axis_index("s")
        wid = sid * NC + cid
        r0 = sid * rpt
        c0 = wid * nch
        e0 = c0 * CHUNK
        pltpu.sync_copy(z_hbm.at[sid], acc_s.at[pl.ds(r0, rpt)])
        pltpu.sync_copy(z_hbm.at[sid], acc_d.at[pl.ds(r0, rpt)])
        pltpu.sync_copy(src_hbm.at[pl.ds(c0, nch)], idxs)
        pltpu.sync_copy(dst_hbm.at[pl.ds(c0, nch)], idxd)
        plsc.subcore_barrier()

        pltpu.async_copy(ea_hbm.at[pl.ds(e0, CHUNK)], rows0, sem0)

        def body(k2, carry):
            i0 = 2 * k2
            i1 = i0 + 1
            pltpu.make_async_copy(
                ea_hbm.at[pl.ds(e0 + i0 * CHUNK, CHUNK)], rows0, sem0).wait()
            pltpu.async_copy(
                ea_hbm.at[pl.ds(e0 + i1 * CHUNK, CHUNK)], rows1, sem1)
            pltpu.sync_copy(rows0, acc_s.at[idxs.at[i0]], add=True)
            pltpu.sync_copy(rows0, acc_d.at[idxd.at[i0]], add=True)
            pltpu.make_async_copy(
                ea_hbm.at[pl.ds(e0 + i1 * CHUNK, CHUNK)], rows1, sem1).wait()

            @pl.when(i0 + 2 < nch)
            def _():
                pltpu.async_copy(
                    ea_hbm.at[pl.ds(e0 + (i0 + 2) * CHUNK, CHUNK)], rows0, sem0)

            pltpu.sync_copy(rows1, acc_s.at[idxs.at[i1]], add=True)
            pltpu.sync_copy(rows1, acc_d.at[idxd.at[i1]], add=True)
            return carry

        lax.fori_loop(0, nch // 2, body, 0)
        if nch % 2:
            i = nch - 1
            pltpu.make_async_copy(
                ea_hbm.at[pl.ds(e0 + i * CHUNK, CHUNK)], rows0, sem0).wait()
            pltpu.sync_copy(rows0, acc_s.at[idxs.at[i]], add=True)
            pltpu.sync_copy(rows0, acc_d.at[idxd.at[i]], add=True)
        plsc.subcore_barrier()
        pltpu.sync_copy(acc_s.at[pl.ds(r0, rpt)], out_hbm.at[cid, 0, sid])
        pltpu.sync_copy(acc_d.at[pl.ds(r0, rpt)], out_hbm.at[cid, 1, sid])

    return k


BN = 2000


def _tc1(accs, x, w1a, w1b):
    N, DF = x.shape
    H = w1a.shape[1]
    grid = (N // BN,)

    def body(a_ref, x_ref, wa_ref, wb_ref, g_ref, dis_ref):
        a = a_ref[...]
        attr = a[0, 0, :, 0:16] + a[0, 1, :, 0:16] + a[1, 0, :, 0:16] + a[1, 1, :, 0:16]
        cnt = a[0, 0, :, 16:17] + a[0, 1, :, 16:17] + a[1, 0, :, 16:17] + a[1, 1, :, 16:17]
        degd = a[0, 1, :, 16:17] + a[1, 1, :, 16:17]
        na = attr / (cnt + 1e-8)
        dis = lax.rsqrt(degd + 1.0)
        h = (jnp.dot(x_ref[...], wa_ref[...], preferred_element_type=f32)
             + jnp.dot(na, wb_ref[...], preferred_element_type=f32))
        g_ref[...] = h * dis
        dis_ref[...] = dis

    return pl.pallas_call(
        body,
        grid=grid,
        in_specs=[
            pl.BlockSpec((NC, 2, BN, 32), lambda i: (0, 0, i, 0)),
            pl.BlockSpec((BN, DF), lambda i: (i, 0)),
            pl.BlockSpec((DF, H), lambda i: (0, 0)),
            pl.BlockSpec((16, H), lambda i: (0, 0)),
        ],
        out_specs=[
            pl.BlockSpec((BN, H), lambda i: (i, 0)),
            pl.BlockSpec((BN, 1), lambda i: (i, 0)),
        ],
        out_shape=[
            jax.ShapeDtypeStruct((N, H), f32),
            jax.ShapeDtypeStruct((N, 1), f32),
        ],
    )(accs, x, w1a, w1b)


def _tc_mid(aggs, g, dis, b, w):
    N, F = g.shape
    Fo = w.shape[1]
    grid = (N // BN,)

    def body(a_ref, g_ref, dis_ref, b_ref, w_ref, out_ref):
        a = a_ref[...]
        dis = dis_ref[...]
        s = dis * (a[0] + a[1] + g_ref[...]) + b_ref[...]
        h = jnp.maximum(s, 0.0)
        out_ref[...] = jnp.dot(h, w_ref[...], preferred_element_type=f32) * dis

    return pl.pallas_call(
        body,
        grid=grid,
        in_specs=[
            pl.BlockSpec((NC, BN, F), lambda i: (0, i, 0)),
            pl.BlockSpec((BN, F), lambda i: (i, 0)),
            pl.BlockSpec((BN, 1), lambda i: (i, 0)),
            pl.BlockSpec((1, F), lambda i: (0, 0)),
            pl.BlockSpec((F, Fo), lambda i: (0, 0)),
        ],
        out_specs=pl.BlockSpec((BN, Fo), lambda i: (i, 0)),
        out_shape=jax.ShapeDtypeStruct((N, Fo), f32),
    )(aggs, g, dis, b, w)


def _tc_relu_scale(aggs, g, dis, b):
    N, F = g.shape
    grid = (N // BN,)

    def body(a_ref, g_ref, dis_ref, b_ref, out_ref):
        a = a_ref[...]
        dis = dis_ref[...]
        z = jnp.maximum(dis * (a[0] + a[1] + g_ref[...]) + b_ref[...], 0.0)
        out_ref[...] = z * dis

    return pl.pallas_call(
        body,
        grid=grid,
        in_specs=[
            pl.BlockSpec((NC, BN, F), lambda i: (0, i, 0)),
            pl.BlockSpec((BN, F), lambda i: (i, 0)),
            pl.BlockSpec((BN, 1), lambda i: (i, 0)),
            pl.BlockSpec((1, F), lambda i: (0, 0)),
        ],
        out_specs=pl.BlockSpec((BN, F), lambda i: (i, 0)),
        out_shape=jax.ShapeDtypeStruct((N, F), f32),
    )(aggs, g, dis, b)


def _tc_mm_relu_scale(aggs, g, dis, b, w):
    N, F = g.shape
    Fo = w.shape[1]
    grid = (N // BN,)

    def body(a_ref, g_ref, dis_ref, b_ref, w_ref, out_ref):
        a = a_ref[...]
        dis = dis_ref[...]
        t = jnp.dot(a[0] + a[1] + g_ref[...], w_ref[...],
                    preferred_element_type=f32)
        h = jnp.maximum(dis * t + b_ref[...], 0.0)
        out_ref[...] = h * dis

    return pl.pallas_call(
        body,
        grid=grid,
        in_specs=[
            pl.BlockSpec((NC, BN, F), lambda i: (0, i, 0)),
            pl.BlockSpec((BN, F), lambda i: (i, 0)),
            pl.BlockSpec((BN, 1), lambda i: (i, 0)),
            pl.BlockSpec((1, Fo), lambda i: (0, 0)),
            pl.BlockSpec((F, Fo), lambda i: (0, 0)),
        ],
        out_specs=pl.BlockSpec((BN, Fo), lambda i: (i, 0)),
        out_shape=jax.ShapeDtypeStruct((N, Fo), f32),
    )(aggs, g, dis, b, w)


def _tc_fin(aggs, g, dis, b, w):
    N, F = g.shape
    Fo = w.shape[1]
    grid = (N // BN,)

    def body(a_ref, g_ref, dis_ref, b_ref, w_ref, out_ref):
        a = a_ref[...]
        t = jnp.dot(a[0] + a[1] + g_ref[...], w_ref[...],
                    preferred_element_type=f32)
        out_ref[...] = dis_ref[...] * t + b_ref[...]

    return pl.pallas_call(
        body,
        grid=grid,
        in_specs=[
            pl.BlockSpec((NC, BN, F), lambda i: (0, i, 0)),
            pl.BlockSpec((BN, F), lambda i: (i, 0)),
            pl.BlockSpec((BN, 1), lambda i: (i, 0)),
            pl.BlockSpec((1, Fo), lambda i: (0, 0)),
            pl.BlockSpec((F, Fo), lambda i: (0, 0)),
        ],
        out_specs=pl.BlockSpec((BN, Fo), lambda i: (i, 0)),
        out_shape=jax.ShapeDtypeStruct((N, Fo), f32),
    )(aggs, g, dis, b, w)


def kernel(x, edge_index, edge_attr, W1, b1, W2, b2, Wd1, bd1, Wd2, bd2):
    N, DF = x.shape
    E = edge_index.shape[1]
    DE = edge_attr.shape[1]
    src = edge_index[0]
    dst = edge_index[1]

    ea_plus = jnp.concatenate(
        [edge_attr, jnp.ones((E, 1), f32), jnp.zeros((E, 32 - DE - 1), f32)],
        axis=1)

    rpt = N // NS
    src2d = src.reshape(E // CHUNK, CHUNK)
    dst2d = dst.reshape(E // CHUNK, CHUNK)
    na_acc = _make_sc_na(N, E)(
        ea_plus, src2d, dst2d, jnp.zeros((NS, rpt, 32), f32)
    ).reshape(NC, 2, N, 32)
    g1, dis = _tc1(na_acc, x, W1[:DF], W1[DF:])

    def conv_agg(g):
        F = g.shape[1]
        return _make_sc_agg(N, E, F)(
            g, src2d, dst2d, jnp.zeros((NS, rpt, F), f32)
        ).reshape(NC, N, F)

    g2 = _tc_mid(conv_agg(g1), g1, dis, b1.reshape(1, -1), W2)
    zd = _tc_relu_scale(conv_agg(g2), g2, dis, b2.reshape(1, -1))
    dhd = _tc_mm_relu_scale(conv_agg(zd), zd, dis, bd1.reshape(1, -1), Wd1)
    return _tc_fin(conv_agg(dhd), dhd, dis, bd2.reshape(1, -1), Wd2)
